# Initial kernel scaffold; baseline (speedup 1.0000x reference)
import jax
import jax.numpy as jnp
from jax.experimental import pallas as pl


def kernel(inputs, pos_table):
    raise NotImplementedError("write your pallas kernel here")


# speedup vs baseline: 1.8060x; 1.8060x over previous
import jax
import jax.numpy as jnp
from jax.experimental import pallas as pl

_BS = 512


def _add_kernel(x_ref, p_ref, o_ref):
    o_ref[...] = x_ref[...] + p_ref[...][None]


def kernel(inputs, pos_table):
    b, s, d = inputs.shape
    grid = (s // _BS,)
    return pl.pallas_call(
        _add_kernel,
        grid=grid,
        in_specs=[
            pl.BlockSpec((b, _BS, d), lambda i: (0, i, 0)),
            pl.BlockSpec((_BS, d), lambda i: (i, 0)),
        ],
        out_specs=pl.BlockSpec((b, _BS, d), lambda i: (0, i, 0)),
        out_shape=jax.ShapeDtypeStruct((b, s, d), inputs.dtype),
    )(inputs, pos_table)


# speedup vs baseline: 1.8085x; 1.0014x over previous
import jax
import jax.numpy as jnp
from jax.experimental import pallas as pl

_BS = 1024


def _add_kernel(x_ref, p_ref, o_ref):
    o_ref[...] = x_ref[...] + p_ref[...][None]


def kernel(inputs, pos_table):
    b, s, d = inputs.shape
    grid = (s // _BS,)
    return pl.pallas_call(
        _add_kernel,
        grid=grid,
        in_specs=[
            pl.BlockSpec((b, _BS, d), lambda i: (0, i, 0)),
            pl.BlockSpec((_BS, d), lambda i: (i, 0)),
        ],
        out_specs=pl.BlockSpec((b, _BS, d), lambda i: (0, i, 0)),
        out_shape=jax.ShapeDtypeStruct((b, s, d), inputs.dtype),
    )(inputs, pos_table)
